# baseline (speedup 1.0000x reference)
import functools

import jax
import jax.numpy as jnp
from jax.experimental import pallas as pl
from jax.experimental.pallas import tpu as pltpu

_TR = 256


def _qmix_block(bb, tr, q_ref, s_ref, w_ref, qw_ref, b2w_ref, b2b_ref,
                out_ref, proj_ref, qexp_ref):
    f32 = jnp.float32
    bf16 = jnp.bfloat16

    s1 = jnp.concatenate(
        [s_ref[...].astype(bf16), jnp.ones((bb, 8), bf16)], axis=1)
    proj_ref[...] = jnp.dot(s1, w_ref[...], preferred_element_type=f32)
    qexp_ref[...] = jnp.dot(q_ref[...].astype(bf16), qw_ref[...],
                            preferred_element_type=f32)

    low = jax.lax.broadcasted_iota(jnp.int32, (tr, 128), 1) < 64
    b2w = b2w_ref[...]
    b2b = b2b_ref[...]

    for t in range(bb // tr):
        rows = pl.ds(t * tr, tr)
        qe = qexp_ref[rows, :]
        y = None
        for j in range(4):
            x = jnp.abs(proj_ref[rows, 128 * j:128 * (j + 1)])
            x = x * qe[:, 128 * j:128 * (j + 1)]
            y = x if y is None else y + x
        hid = y + pltpu.roll(y, 64, axis=1)

        bw = proj_ref[rows, 512:640]
        r = pltpu.roll(bw, 64, axis=1)
        b1d = jnp.where(low, bw, r)
        w2d = jnp.abs(jnp.where(low, r, bw))

        mixed = hid + b1d
        mixed = jnp.where(mixed > 0.0, mixed,
                          jnp.exp(jnp.minimum(mixed, 0.0)) - 1.0)

        h2 = jnp.maximum(proj_ref[rows, 640:768], 0.0)
        z = mixed * w2d + h2 * b2w
        out_ref[rows, :] = jnp.sum(z, axis=1, keepdims=True) + b2b


def kernel(qagents, state, w_cat, expand, reduce, b2w, b2b):
    del reduce
    f32 = jnp.float32
    B, A = qagents.shape
    S = state.shape[1]
    H = 64
    w0 = A * H

    packed = jnp.concatenate([
        w_cat[:, 0:w0],
        w_cat[:, w0:w0 + H],
        0.5 * w_cat[:, w0 + 128:w0 + 128 + H],
        w_cat[:, w0 + 256:w0 + 256 + S],
    ], axis=1)
    w_bf = jnp.concatenate(
        [packed, jnp.zeros((135 - S, 768), f32)], axis=0).astype(jnp.bfloat16)
    qw_bf = expand.astype(jnp.bfloat16)

    BB = 4096 if B % 4096 == 0 else max(8, ((B + 7) // 8) * 8)
    TR = _TR if BB % _TR == 0 else BB
    grid_b = pl.cdiv(B, BB)
    b_pad = grid_b * BB
    if b_pad != B:
        qagents = jnp.pad(qagents, ((0, b_pad - B), (0, 0)))
        state = jnp.pad(state, ((0, b_pad - B), (0, 0)))

    out = pl.pallas_call(
        functools.partial(_qmix_block, BB, TR),
        out_shape=jax.ShapeDtypeStruct((b_pad, 1), f32),
        grid=(grid_b,),
        in_specs=[
            pl.BlockSpec((BB, A), lambda i: (i, 0)),
            pl.BlockSpec((BB, S), lambda i: (i, 0)),
            pl.BlockSpec((136, 768), lambda i: (0, 0)),
            pl.BlockSpec((A, w0), lambda i: (0, 0)),
            pl.BlockSpec((1, 128), lambda i: (0, 0)),
            pl.BlockSpec((1, 1), lambda i: (0, 0)),
        ],
        out_specs=pl.BlockSpec((BB, 1), lambda i: (i, 0)),
        scratch_shapes=[
            pltpu.VMEM((BB, 768), f32),
            pltpu.VMEM((BB, 512), f32),
        ],
        compiler_params=pltpu.CompilerParams(
            dimension_semantics=("parallel",)),
    )(qagents, state, w_bf, qw_bf, b2w, b2b)
    return out.reshape(-1)[:B]


# speedup vs baseline: 1.2628x; 1.2628x over previous
import functools

import jax
import jax.numpy as jnp
from jax.experimental import pallas as pl
from jax.experimental.pallas import tpu as pltpu

_TR = 256


def _qmix_block(bb, tr, q_ref, s_ref, w_ref, qw_ref, rw_ref, b2b_ref, out_ref):
    f32 = jnp.float32
    bf16 = jnp.bfloat16

    s1 = jnp.concatenate(
        [s_ref[...].astype(bf16), jnp.ones((bb, 8), bf16)], axis=1)
    q1 = q_ref[...].astype(bf16)
    low = jax.lax.broadcasted_iota(jnp.int32, (tr, 128), 1) < 64
    b2b = b2b_ref[...]

    for t in range(bb // tr):
        r0, r1 = t * tr, (t + 1) * tr
        p = jnp.dot(s1[r0:r1, :], w_ref[...], preferred_element_type=f32)
        qe = jnp.dot(q1[r0:r1, :], qw_ref[...], preferred_element_type=f32)

        y = None
        for j in range(4):
            x = jnp.abs(p[:, 128 * j:128 * (j + 1)])
            x = x * qe[:, 128 * j:128 * (j + 1)]
            y = x if y is None else y + x
        hid = y + pltpu.roll(y, 64, axis=1)

        bw = p[:, 512:640]
        r = pltpu.roll(bw, 64, axis=1)
        b1d = jnp.where(low, bw, r)
        w2d = jnp.abs(jnp.where(low, r, bw))

        mixed = hid + b1d
        mixed = jnp.where(mixed > 0.0, mixed,
                          jnp.exp(jnp.minimum(mixed, 0.0)) - 1.0)

        h2 = jnp.maximum(p[:, 640:768], 0.0)
        zc = jnp.concatenate([mixed * w2d, h2], axis=1).astype(bf16)
        res = jnp.dot(zc, rw_ref[...], preferred_element_type=f32)
        out_ref[r0:r1, :] = res[:, 0:1] + b2b


def kernel(qagents, state, w_cat, expand, reduce, b2w, b2b):
    del reduce
    f32 = jnp.float32
    B, A = qagents.shape
    S = state.shape[1]
    H = 64
    w0 = A * H

    packed = jnp.concatenate([
        w_cat[:, 0:w0],
        w_cat[:, w0:w0 + H],
        0.5 * w_cat[:, w0 + 128:w0 + 128 + H],
        w_cat[:, w0 + 256:w0 + 256 + S],
    ], axis=1)
    w_bf = jnp.concatenate(
        [packed, jnp.zeros((135 - S, 768), f32)], axis=0).astype(jnp.bfloat16)
    qw_bf = expand.astype(jnp.bfloat16)
    rw = jnp.zeros((256, 8), f32)
    rw = rw.at[0:128, 0].set(1.0).at[128:256, 0].set(b2w[0, :])
    rw_bf = rw.astype(jnp.bfloat16)

    BB = 4096 if B % 4096 == 0 else max(8, ((B + 7) // 8) * 8)
    TR = _TR if BB % _TR == 0 else BB
    grid_b = pl.cdiv(B, BB)
    b_pad = grid_b * BB
    if b_pad != B:
        qagents = jnp.pad(qagents, ((0, b_pad - B), (0, 0)))
        state = jnp.pad(state, ((0, b_pad - B), (0, 0)))

    out = pl.pallas_call(
        functools.partial(_qmix_block, BB, TR),
        out_shape=jax.ShapeDtypeStruct((b_pad, 1), f32),
        grid=(grid_b,),
        in_specs=[
            pl.BlockSpec((BB, A), lambda i: (i, 0)),
            pl.BlockSpec((BB, S), lambda i: (i, 0)),
            pl.BlockSpec((136, 768), lambda i: (0, 0)),
            pl.BlockSpec((A, w0), lambda i: (0, 0)),
            pl.BlockSpec((256, 8), lambda i: (0, 0)),
            pl.BlockSpec((1, 1), lambda i: (0, 0)),
        ],
        out_specs=pl.BlockSpec((BB, 1), lambda i: (i, 0)),
        compiler_params=pltpu.CompilerParams(
            dimension_semantics=("parallel",)),
    )(qagents, state, w_bf, qw_bf, rw_bf, b2b)
    return out.reshape(-1)[:B]
